# Initial kernel scaffold; baseline (speedup 1.0000x reference)
import jax
import jax.numpy as jnp
from jax.experimental import pallas as pl


def kernel(handler_parameters, handler_parameter_ids, parameter_delta):
    raise NotImplementedError("write your pallas kernel here")


# speedup vs baseline: 2.1103x; 2.1103x over previous
import functools

import jax
import jax.numpy as jnp
from jax import lax
from jax.experimental import pallas as pl
from jax.experimental.pallas import tpu as pltpu
from jax.experimental.pallas import tpu_sc as plsc

N_SMIRKS = 64
N_ATTRS = 2
M = 8388608
FLOATS = M * N_ATTRS

NC, NS, L = 2, 16, 16
NW = NC * NS
ROWS_W = M // NW
R = 16384
CF = R * N_ATTRS
NCHUNK = ROWS_W // R

_mesh = plsc.VectorSubcoreMesh(core_axis_name="c", subcore_axis_name="s")


@functools.partial(
    pl.kernel,
    out_type=jax.ShapeDtypeStruct((FLOATS,), jnp.float32),
    mesh=_mesh,
    compiler_params=pltpu.CompilerParams(needs_layout_passes=False),
    scratch_types=[
        pltpu.VMEM((N_SMIRKS * N_ATTRS,), jnp.float32),
        pltpu.VMEM((R,), jnp.int32),
        pltpu.VMEM((CF,), jnp.float32),
        pltpu.VMEM((CF,), jnp.float32),
    ],
)
def _sc_add_delta(hp_hbm, ids_hbm, delta_hbm, out_hbm,
                  delta_v, ids_v, hp_v, out_v):
    wid = lax.axis_index("s") * NC + lax.axis_index("c")
    pltpu.sync_copy(delta_hbm, delta_v)
    iota = lax.iota(jnp.int32, L)
    half_iota = iota // 2
    parity = iota % 2
    row0 = wid * ROWS_W

    def chunk_body(c, carry):
        r0 = row0 + c * R
        f0 = r0 * N_ATTRS
        pltpu.sync_copy(ids_hbm.at[pl.ds(r0, R)], ids_v)
        pltpu.sync_copy(hp_hbm.at[pl.ds(f0, CF)], hp_v)

        def vec_body(i, carry2):
            v_row = half_iota + i * 8
            v_ids = plsc.load_gather(ids_v, [v_row])
            v_didx = v_ids * 2 + parity
            v_d = plsc.load_gather(delta_v, [v_didx])
            out_v[pl.ds(i * L, L)] = hp_v[pl.ds(i * L, L)] + v_d
            return carry2

        lax.fori_loop(0, CF // L, vec_body, 0)
        pltpu.sync_copy(out_v, out_hbm.at[pl.ds(f0, CF)])
        return carry

    lax.fori_loop(0, NCHUNK, chunk_body, 0)


def kernel(handler_parameters, handler_parameter_ids, parameter_delta):
    hp_flat = handler_parameters.reshape(FLOATS)
    out_flat = _sc_add_delta(hp_flat, handler_parameter_ids, parameter_delta)
    return out_flat.reshape(M, N_ATTRS)


# speedup vs baseline: 2.1537x; 1.0205x over previous
import functools

import jax
import jax.numpy as jnp
from jax import lax
from jax.experimental import pallas as pl
from jax.experimental.pallas import tpu as pltpu
from jax.experimental.pallas import tpu_sc as plsc

N_SMIRKS = 64
N_ATTRS = 2
M = 8388608
FLOATS = M * N_ATTRS

NC, NS, L = 2, 16, 16
NW = NC * NS
ROWS_W = M // NW
R = 16384
CF = R * N_ATTRS
NCHUNK = ROWS_W // R

_mesh = plsc.VectorSubcoreMesh(core_axis_name="c", subcore_axis_name="s")


@functools.partial(
    pl.kernel,
    out_type=jax.ShapeDtypeStruct((FLOATS,), jnp.float32),
    mesh=_mesh,
    compiler_params=pltpu.CompilerParams(needs_layout_passes=False),
    scratch_types=[
        pltpu.VMEM((N_SMIRKS * N_ATTRS,), jnp.float32),
        pltpu.VMEM((R,), jnp.int32),
        pltpu.VMEM((CF,), jnp.float32),
        pltpu.VMEM((CF,), jnp.float32),
    ],
)
def _sc_add_delta(hp_hbm, ids_hbm, delta_hbm, out_hbm,
                  delta_v, ids_v, hp_v, out_v):
    wid = lax.axis_index("s") * NC + lax.axis_index("c")
    pltpu.sync_copy(delta_hbm, delta_v)
    iota = lax.iota(jnp.int32, L)
    half_iota = iota // 2
    parity = iota % 2
    row0 = wid * ROWS_W

    def chunk_body(c, carry):
        r0 = row0 + c * R
        f0 = r0 * N_ATTRS
        pltpu.sync_copy(ids_hbm.at[pl.ds(r0, R)], ids_v)
        pltpu.sync_copy(hp_hbm.at[pl.ds(f0, CF)], hp_v)

        @plsc.parallel_loop(0, CF // L, unroll=8)
        def vec_body(i):
            v_row = half_iota + i * 8
            v_ids = plsc.load_gather(ids_v, [v_row])
            v_didx = v_ids * 2 + parity
            v_d = plsc.load_gather(delta_v, [v_didx])
            out_v[pl.ds(i * L, L)] = hp_v[pl.ds(i * L, L)] + v_d
        pltpu.sync_copy(out_v, out_hbm.at[pl.ds(f0, CF)])
        return carry

    lax.fori_loop(0, NCHUNK, chunk_body, 0)


def kernel(handler_parameters, handler_parameter_ids, parameter_delta):
    hp_flat = handler_parameters.reshape(FLOATS)
    out_flat = _sc_add_delta(hp_flat, handler_parameter_ids, parameter_delta)
    return out_flat.reshape(M, N_ATTRS)
